# baseline (speedup 1.0000x reference)
import functools

import jax
import jax.numpy as jnp
from jax import lax
from jax.experimental import pallas as pl
from jax.experimental.pallas import tpu as pltpu
from jax.experimental.pallas import tpu_sc as plsc

NC = 2
NS = 16
NW = NC * NS
LANES = 16
CH = 40
PB = 128
HALF = 50048


def _repack(tables_t, N, Dm, V):
  nj = HALF // PB

  def body(in1, in2, o):
    x1 = in1[0]
    x2 = in2[0]
    eye = jnp.eye(Dm, dtype=jnp.float32)
    dn = (((0,), (0,)), ((), ()))
    t1 = lax.dot_general(x1, eye, dn, preferred_element_type=jnp.float32)
    t2 = lax.dot_general(x2, eye, dn, preferred_element_type=jnp.float32)
    o[...] = jnp.concatenate([t1, t2], axis=1)

  return pl.pallas_call(
      body,
      grid=(N, nj),
      in_specs=[
          pl.BlockSpec((1, Dm, PB), lambda i, j: (i, 0, j)),
          pl.BlockSpec((1, Dm, PB), lambda i, j: (i, 0, nj + j)),
      ],
      out_specs=pl.BlockSpec((PB, 2 * Dm), lambda i, j: (i * nj + j, 0)),
      out_shape=jax.ShapeDtypeStruct((N * HALF, 2 * Dm), jnp.float32),
  )(tables_t, tables_t)


def _build(B, N, L, D):
  assert B % NW == 0 and L % CH == 0 and D % LANES == 0
  b_per_w = B // NW
  cpb = L // CH
  M = B * L
  mesh = plsc.VectorSubcoreMesh(core_axis_name="c", subcore_axis_name="s")

  full = [s0 for s0 in range(0, L - LANES + 1, LANES)]
  if L % LANES:
    full.append(L - LANES)

  @functools.partial(
      pl.kernel,
      out_type=jax.ShapeDtypeStruct((M, D), jnp.float32),
      mesh=mesh,
      compiler_params=pltpu.CompilerParams(needs_layout_passes=False),
      scratch_types=[
          pltpu.VMEM((N, L), jnp.int32),
          pltpu.VMEM((N * L,), jnp.int32),
          pltpu.VMEM((N * L,), jnp.int32),
          pltpu.VMEM((CH, D), jnp.float32),
      ]
      + [pltpu.VMEM((CH, 2 * D), jnp.float32) for _ in range(N)]
      + [pltpu.SemaphoreType.DMA],
  )
  def embed(codes_hbm, tabp_hbm, out_hbm, idx_raw, idx_gat, sel, out_v, *rest):
    bufs = rest[:N]
    sem = rest[N]
    wid = lax.axis_index("s") * NC + lax.axis_index("c")

    def batch_body(bb, carry):
      b = wid * b_per_w + bb
      pltpu.sync_copy(codes_hbm.at[b], idx_raw)

      half = jnp.full((LANES,), HALF, jnp.int32)
      zero = jnp.zeros((LANES,), jnp.int32)
      for i in range(N):
        base = jnp.full((LANES,), i * HALF, jnp.int32)
        for s0 in full:
          c = idx_raw[i, pl.ds(s0, LANES)]
          hi = c >= half
          adj = jnp.where(hi, c - half, c)
          idx_gat[pl.ds(i * L + s0, LANES)] = base + adj
          sel[pl.ds(i * L + s0, LANES)] = jnp.where(
              hi, jnp.full((LANES,), 1, jnp.int32), zero)

      def chunk_body(cidx, carry2):
        l0 = cidx * CH
        copies = [
            pltpu.async_copy(
                tabp_hbm.at[idx_gat.at[pl.ds(i * L + l0, CH)]], bufs[i], sem)
            for i in range(N)
        ]
        for cp in copies:
          cp.wait()

        def sum_body(r, carry3):
          masks = []
          for i in range(N):
            pos = jnp.full((LANES,), i * L, jnp.int32) + (l0 + r)
            pv = plsc.load_gather(sel, [pos])
            masks.append(pv == 1)
          for d in range(D // LANES):
            lo = pl.ds(d * LANES, LANES)
            hi_s = pl.ds(D + d * LANES, LANES)
            acc = jnp.where(masks[0], bufs[0][r, hi_s], bufs[0][r, lo])
            for i in range(1, N):
              acc = acc + jnp.where(masks[i], bufs[i][r, hi_s], bufs[i][r, lo])
            out_v[r, lo] = acc
          return carry3

        lax.fori_loop(0, CH, sum_body, 0)
        pltpu.sync_copy(out_v, out_hbm.at[pl.ds(b * L + l0, CH)])
        return carry2

      lax.fori_loop(0, cpb, chunk_body, 0)
      return carry

    lax.fori_loop(0, b_per_w, batch_body, 0)

  return embed


def kernel(codes, tables):
  B, N, L = codes.shape
  V = tables.shape[1]
  D = tables.shape[2]
  assert V <= 2 * HALF and HALF % PB == 0
  tables_t = jnp.transpose(tables, (0, 2, 1))
  tabp = _repack(tables_t, N, D, V)
  out2d = _build(B, N, L, D)(codes, tabp)
  return out2d.reshape(B, L, D)


# speedup vs baseline: 2.5705x; 2.5705x over previous
import functools

import jax
import jax.numpy as jnp
from jax import lax
from jax.experimental import pallas as pl
from jax.experimental.pallas import tpu as pltpu
from jax.experimental.pallas import tpu_sc as plsc

NC = 2
NS = 16
NW = NC * NS
LANES = 16
CH = 40
PB = 128
HALF = 50048


def _repack(tables_t, N, Dm, V):
  LB = 2176
  nj = HALF // LB

  def body(in1, in2, o):
    x1 = in1[0]
    x2 = in2[0]
    o[...] = jnp.concatenate([x1.T, x2.T], axis=1)

  return pl.pallas_call(
      body,
      grid=(N, nj),
      in_specs=[
          pl.BlockSpec((1, Dm, LB), lambda i, j: (i, 0, j)),
          pl.BlockSpec((1, Dm, LB), lambda i, j: (i, 0, nj + j)),
      ],
      out_specs=pl.BlockSpec((LB, 2 * Dm), lambda i, j: (i * nj + j, 0)),
      out_shape=jax.ShapeDtypeStruct((N * HALF, 2 * Dm), jnp.float32),
  )(tables_t, tables_t)


def _build(B, N, L, D):
  assert B % NW == 0 and L % CH == 0 and D % LANES == 0
  b_per_w = B // NW
  cpb = L // CH
  M = B * L
  mesh = plsc.VectorSubcoreMesh(core_axis_name="c", subcore_axis_name="s")

  full = [s0 for s0 in range(0, L - LANES + 1, LANES)]
  if L % LANES:
    full.append(L - LANES)

  @functools.partial(
      pl.kernel,
      out_type=jax.ShapeDtypeStruct((M, D), jnp.float32),
      mesh=mesh,
      compiler_params=pltpu.CompilerParams(needs_layout_passes=False),
      scratch_types=[
          pltpu.VMEM((N, L), jnp.int32),
          pltpu.VMEM((N * L,), jnp.int32),
          pltpu.VMEM((N * L,), jnp.int32),
          pltpu.VMEM((CH, D), jnp.float32),
      ]
      + [pltpu.VMEM((CH, 2 * D), jnp.float32) for _ in range(N)]
      + [pltpu.SemaphoreType.DMA],
  )
  def embed(codes_hbm, tabp_hbm, out_hbm, idx_raw, idx_gat, sel, out_v, *rest):
    bufs = rest[:N]
    sem = rest[N]
    wid = lax.axis_index("s") * NC + lax.axis_index("c")

    def batch_body(bb, carry):
      b = wid * b_per_w + bb
      pltpu.sync_copy(codes_hbm.at[b], idx_raw)

      half = jnp.full((LANES,), HALF, jnp.int32)
      zero = jnp.zeros((LANES,), jnp.int32)
      for i in range(N):
        base = jnp.full((LANES,), i * HALF, jnp.int32)
        for s0 in full:
          c = idx_raw[i, pl.ds(s0, LANES)]
          hi = c >= half
          adj = jnp.where(hi, c - half, c)
          idx_gat[pl.ds(i * L + s0, LANES)] = base + adj
          sel[pl.ds(i * L + s0, LANES)] = jnp.where(
              hi, jnp.full((LANES,), 1, jnp.int32), zero)

      def chunk_body(cidx, carry2):
        l0 = cidx * CH
        copies = [
            pltpu.async_copy(
                tabp_hbm.at[idx_gat.at[pl.ds(i * L + l0, CH)]], bufs[i], sem)
            for i in range(N)
        ]
        for cp in copies:
          cp.wait()

        def sum_body(r, carry3):
          masks = []
          for i in range(N):
            pos = jnp.full((LANES,), i * L, jnp.int32) + (l0 + r)
            pv = plsc.load_gather(sel, [pos])
            masks.append(pv == 1)
          for d in range(D // LANES):
            lo = pl.ds(d * LANES, LANES)
            hi_s = pl.ds(D + d * LANES, LANES)
            acc = jnp.where(masks[0], bufs[0][r, hi_s], bufs[0][r, lo])
            for i in range(1, N):
              acc = acc + jnp.where(masks[i], bufs[i][r, hi_s], bufs[i][r, lo])
            out_v[r, lo] = acc
          return carry3

        lax.fori_loop(0, CH, sum_body, 0)
        pltpu.sync_copy(out_v, out_hbm.at[pl.ds(b * L + l0, CH)])
        return carry2

      lax.fori_loop(0, cpb, chunk_body, 0)
      return carry

    lax.fori_loop(0, b_per_w, batch_body, 0)

  return embed


def kernel(codes, tables):
  B, N, L = codes.shape
  V = tables.shape[1]
  D = tables.shape[2]
  assert V <= 2 * HALF and HALF % PB == 0
  tables_t = jnp.transpose(tables, (0, 2, 1))
  tabp = _repack(tables_t, N, D, V)
  out2d = _build(B, N, L, D)(codes, tabp)
  return out2d.reshape(B, L, D)
